# baseline (speedup 1.0000x reference)
import jax
import jax.numpy as jnp
from jax import lax
from jax.experimental import pallas as pl
from jax.experimental.pallas import tpu as pltpu
from jax.experimental.pallas import tpu_sc as plsc

_B = 16384
_D = 32
_NC, _NS = 2, 16
_NW = _NC * _NS
_BPW = _B // _NW
_CH = 128
_NCH = _BPW // _CH

_EPS = 1e-3


def _gather_body(ut_hbm, mt_hbm, uidx_hbm, midx_hbm, uout_hbm, mout_hbm,
                 uidx_v, midx_v, urows_v, mrows_v, sem):
    wid = lax.axis_index("s") * _NC + lax.axis_index("c")
    pltpu.sync_copy(uidx_hbm.at[wid], uidx_v)
    pltpu.sync_copy(midx_hbm.at[wid], midx_v)
    cps = []
    for j in range(_NCH):
        cps.append(pltpu.async_copy(ut_hbm.at[uidx_v.at[j]], urows_v.at[j], sem))
        cps.append(pltpu.async_copy(mt_hbm.at[midx_v.at[j]], mrows_v.at[j], sem))
    for c in cps:
        c.wait()
    pltpu.sync_copy(urows_v, uout_hbm.at[wid])
    pltpu.sync_copy(mrows_v, mout_hbm.at[wid])


def _bn(x, g, b):
    mu = jnp.mean(x, axis=0, keepdims=True)
    var = jnp.mean(jnp.square(x - mu), axis=0, keepdims=True)
    return g * (x - mu) * lax.rsqrt(var + _EPS) + b


def _mlp_body(u_ref, m_ref,
              W1r, b1r, g1r, be1r,
              W2r, b2r, g2r, be2r,
              W3ur, W3mr, b3r, g3r, be3r,
              W4r, b4r, g4r, be4r,
              W5r, b5r, g5r, be5r,
              Wor, bor, o_ref):
    f32 = jnp.float32
    u = jnp.maximum(jnp.dot(u_ref[:], W1r[:], preferred_element_type=f32) + b1r[:], 0.0)
    u = _bn(u, g1r[:], be1r[:])
    m = jnp.maximum(jnp.dot(m_ref[:], W2r[:], preferred_element_type=f32) + b2r[:], 0.0)
    m = _bn(m, g2r[:], be2r[:])
    x = (jnp.dot(u, W3ur[:], preferred_element_type=f32)
         + jnp.dot(m, W3mr[:], preferred_element_type=f32) + b3r[:])
    x = _bn(jnp.maximum(x, 0.0), g3r[:], be3r[:])
    x = jnp.maximum(jnp.dot(x, W4r[:], preferred_element_type=f32) + b4r[:], 0.0)
    x = _bn(x, g4r[:], be4r[:])
    x = jnp.maximum(jnp.dot(x, W5r[:], preferred_element_type=f32) + b5r[:], 0.0)
    x = _bn(x, g5r[:], be5r[:])
    o_ref[:] = jax.nn.sigmoid(jnp.dot(x, Wor[:], preferred_element_type=f32) + bor[:])


def kernel(inputs, user_table, movie_table,
           W1, b1, g1, be1,
           W2, b2, g2, be2,
           W3, b3, g3, be3,
           W4, b4, g4, be4,
           W5, b5, g5, be5,
           Wo, bo):
    uidx = inputs[:, 0].reshape(_NW, _NCH, _CH)
    midx = inputs[:, 1].reshape(_NW, _NCH, _CH)

    mesh = plsc.VectorSubcoreMesh(core_axis_name="c", subcore_axis_name="s")
    gathered = pl.kernel(
        _gather_body,
        out_type=[jax.ShapeDtypeStruct((_NW, _NCH, _CH, _D), jnp.float32),
                  jax.ShapeDtypeStruct((_NW, _NCH, _CH, _D), jnp.float32)],
        mesh=mesh,
        scratch_types=[
            pltpu.VMEM((_NCH, _CH), jnp.int32),
            pltpu.VMEM((_NCH, _CH), jnp.int32),
            pltpu.VMEM((_NCH, _CH, _D), jnp.float32),
            pltpu.VMEM((_NCH, _CH, _D), jnp.float32),
            pltpu.SemaphoreType.DMA,
        ],
        compiler_params=pltpu.CompilerParams(use_tc_tiling_on_sc=False),
    )(user_table, movie_table, uidx, midx)
    u_emb = gathered[0].reshape(_B, _D)
    m_emb = gathered[1].reshape(_B, _D)

    H2 = W1.shape[1]
    out = pl.pallas_call(
        _mlp_body,
        out_shape=jax.ShapeDtypeStruct((_B, 1), jnp.float32),
    )(u_emb, m_emb,
      W1, b1.reshape(1, -1), g1.reshape(1, -1), be1.reshape(1, -1),
      W2, b2.reshape(1, -1), g2.reshape(1, -1), be2.reshape(1, -1),
      W3[:H2], W3[H2:], b3.reshape(1, -1), g3.reshape(1, -1), be3.reshape(1, -1),
      W4, b4.reshape(1, -1), g4.reshape(1, -1), be4.reshape(1, -1),
      W5, b5.reshape(1, -1), g5.reshape(1, -1), be5.reshape(1, -1),
      Wo, bo.reshape(1, -1))
    return out
